# baseline (speedup 1.0000x reference)
import functools

import jax
import jax.numpy as jnp
from jax import lax
from jax.experimental import pallas as pl
from jax.experimental.pallas import tpu as pltpu
from jax.experimental.pallas import tpu_sc as plsc

N = 10000
E = 320000
D = 128
NPAD = 10240
NC = 2
NS = 16
C = 64
NSL = 5
NCH_A = 160
NCH_B = 320
EPA = NCH_A * C * NS * NC
EPB = NCH_B * C * NS
EC = EPB // 32
DRN = NPAD // NS
HD = DRN // 16
RB = 1000
NB = N // RB
BN_EPS = 1e-5


def _sc_gather_scatter_body(table, gidx, sidx, zrows, out, acc, *scr):
    gv = scr[0:NSL]
    sv = scr[NSL:2 * NSL]
    rv = scr[2 * NSL:3 * NSL]
    dv = scr[3 * NSL]
    sem_g = scr[3 * NSL + 1:4 * NSL + 1]
    sem_s = scr[4 * NSL + 1:5 * NSL + 1]
    sem_r = scr[5 * NSL + 1:6 * NSL + 1]
    sem_c = scr[6 * NSL + 1:7 * NSL + 1]
    nch = gidx.shape[2]
    c = lax.axis_index("c")
    s = lax.axis_index("s")
    pltpu.sync_copy(zrows, dv)
    for q in range(DRN // HD):
        pltpu.sync_copy(dv, acc.at[pl.ds(s * DRN + q * HD, HD)])
    plsc.subcore_barrier()

    for i in range(NSL):
        pltpu.async_copy(gidx.at[c, s, i], gv[i], sem_g[i])
        pltpu.async_copy(sidx.at[c, s, i], sv[i], sem_s[i])
    for i in range(NSL):
        pltpu.make_async_copy(gidx.at[c, s, i], gv[i], sem_g[i]).wait()
        pltpu.async_copy(table.at[gv[i]], rv[i], sem_r[i])

    def group(t, carry):
        j = NSL * t
        for i in range(NSL):
            pltpu.make_async_copy(table.at[gv[i]], rv[i], sem_r[i]).wait()
            pltpu.make_async_copy(sidx.at[c, s, j + i], sv[i], sem_s[i]).wait()
            pltpu.async_copy(rv[i], acc.at[sv[i]], sem_c[i], add=True)
        for i in range(NSL):
            pltpu.make_async_copy(rv[i], acc.at[sv[i]], sem_c[i]).wait()

            @pl.when(t < nch // NSL - 1)
            def _():
                pltpu.async_copy(gidx.at[c, s, j + NSL + i], gv[i], sem_g[i])
                pltpu.async_copy(sidx.at[c, s, j + NSL + i], sv[i], sem_s[i])
                pltpu.make_async_copy(
                    gidx.at[c, s, j + NSL + i], gv[i], sem_g[i]).wait()
                pltpu.async_copy(table.at[gv[i]], rv[i], sem_r[i])

        return carry

    lax.fori_loop(0, nch // NSL, group, 0)
    plsc.subcore_barrier()

    for q in range(DRN // HD):
        pltpu.sync_copy(acc.at[pl.ds(s * DRN + q * HD, HD)], dv)
        pltpu.sync_copy(dv, out.at[c, pl.ds(s * DRN + q * HD, HD)])


def _sc_counts_body(sidx, didx, out_s, out_d, sv, dv, cs, cd):
    c = lax.axis_index("c")
    s = lax.axis_index("s")
    wid = c * NS + s
    pltpu.sync_copy(sidx.at[wid], sv)
    pltpu.sync_copy(didx.at[wid], dv)

    zero = jnp.zeros((16,), jnp.float32)

    def zbody(i, carry):
        cs[pl.ds(i * 16, 16)] = zero
        cd[pl.ds(i * 16, 16)] = zero
        return carry

    lax.fori_loop(0, NPAD // 16, zbody, 0)

    ones = jnp.ones((16,), jnp.float32)

    def body(i, carry):
        iv = sv[pl.ds(i * 16, 16)]
        plsc.addupdate_scatter(cs, [iv], ones)
        jv = dv[pl.ds(i * 16, 16)]
        plsc.addupdate_scatter(cd, [jv], ones)
        return carry

    lax.fori_loop(0, EC // 16, body, 0)
    pltpu.sync_copy(cs, out_s.at[wid])
    pltpu.sync_copy(cd, out_d.at[wid])


@functools.lru_cache(maxsize=None)
def _sc_kernels():
    mesh = plsc.VectorSubcoreMesh(
        core_axis_name="c", subcore_axis_name="s",
        num_cores=NC, num_subcores=NS)
    params = pltpu.CompilerParams(needs_layout_passes=False)

    def make_gs(nch):
        return pl.kernel(
            _sc_gather_scatter_body,
            out_type=jax.ShapeDtypeStruct((NC, NPAD, D), jnp.float32),
            mesh=mesh,
            scratch_types=(
                [pltpu.VMEM_SHARED((NPAD, D), jnp.float32)]
                + [pltpu.VMEM((C,), jnp.int32)] * NSL
                + [pltpu.VMEM((C,), jnp.int32)] * NSL
                + [pltpu.VMEM((C, D), jnp.float32)] * NSL
                + [pltpu.VMEM((HD, D), jnp.float32)]
                + [pltpu.SemaphoreType.DMA] * (4 * NSL)
            ),
            compiler_params=params,
        )

    counts = pl.kernel(
        _sc_counts_body,
        out_type=(jax.ShapeDtypeStruct((NC * NS, NPAD), jnp.float32),
                  jax.ShapeDtypeStruct((NC * NS, NPAD), jnp.float32)),
        mesh=mesh,
        scratch_types=[
            pltpu.VMEM((EC,), jnp.int32),
            pltpu.VMEM((EC,), jnp.int32),
            pltpu.VMEM((NPAD,), jnp.float32),
            pltpu.VMEM((NPAD,), jnp.float32),
        ],
        compiler_params=params,
    )
    return make_gs(NCH_A), make_gs(NCH_B), counts


def _k0_body(csp, cdp, dinv_o, cntr_o, cnt_o):
    cntr = jnp.sum(csp[...], axis=0)
    deg = jnp.sum(cdp[...], axis=0) + 1.0
    dinv_o[...] = lax.rsqrt(deg)[:, None]
    cntr_o[...] = cntr[:, None]
    cnt_o[...] = jnp.maximum(cntr, 1.0)[:, None]


_k0 = pl.pallas_call(
    _k0_body,
    out_shape=(jax.ShapeDtypeStruct((NPAD, 1), jnp.float32),) * 3,
)


def _k1_body(x, w, dinv, xs_o, xw_o):
    xw = jnp.dot(x[...], w[...], preferred_element_type=jnp.float32)
    xw_o[...] = xw
    xs_o[...] = xw * dinv[...]


_k1 = pl.pallas_call(
    _k1_body,
    grid=(NB,),
    in_specs=[
        pl.BlockSpec((RB, D), lambda i: (i, 0)),
        pl.BlockSpec((D, D), lambda i: (0, 0)),
        pl.BlockSpec((RB, 1), lambda i: (i, 0)),
    ],
    out_specs=(
        pl.BlockSpec((RB, D), lambda i: (i, 0)),
        pl.BlockSpec((RB, D), lambda i: (i, 0)),
    ),
    out_shape=(jax.ShapeDtypeStruct((N, D), jnp.float32),) * 2,
)


def _k2_body(agg2, xw, dinv, b, hh_o):
    di = dinv[...]
    a = agg2[0] + agg2[1]
    h = jnp.maximum(di * a + di * di * xw[...] + b[...], 0.0)
    hh_o[0] = h
    hh_o[1] = h * h


_k2 = pl.pallas_call(
    _k2_body,
    grid=(NB,),
    in_specs=[
        pl.BlockSpec((NC, RB, D), lambda i: (0, i, 0)),
        pl.BlockSpec((RB, D), lambda i: (i, 0)),
        pl.BlockSpec((RB, 1), lambda i: (i, 0)),
        pl.BlockSpec((1, D), lambda i: (0, 0)),
    ],
    out_specs=pl.BlockSpec((2, RB, D), lambda i: (0, i, 0)),
    out_shape=jax.ShapeDtypeStruct((2, N, D), jnp.float32),
)


def _k3_body(s12, hh, x, cntr, cnt, xn_o):
    h = hh[0]
    xb = x[...]
    s = cntr[...] * h * h - 2.0 * h * s12[0] + s12[1]
    gg = jnp.tanh(s / cnt[...])
    xn_o[...] = (1.0 - gg) * xb + gg * h


_k3 = pl.pallas_call(
    _k3_body,
    grid=(NB,),
    in_specs=[
        pl.BlockSpec((NC, RB, D), lambda i: (0, i, 0)),
        pl.BlockSpec((2, RB, D), lambda i: (0, i, 0)),
        pl.BlockSpec((RB, D), lambda i: (i, 0)),
        pl.BlockSpec((RB, 1), lambda i: (i, 0)),
        pl.BlockSpec((RB, 1), lambda i: (i, 0)),
    ],
    out_specs=pl.BlockSpec((RB, D), lambda i: (i, 0)),
    out_shape=jax.ShapeDtypeStruct((N, D), jnp.float32),
)


def _k4_body(x0, x1, x2, x3, ssum_o, ssq_o):
    i = pl.program_id(0)

    @pl.when(i == 0)
    def _():
        ssum_o[...] = jnp.zeros((4, D), jnp.float32)
        ssq_o[...] = jnp.zeros((4, D), jnp.float32)

    for j, xr in enumerate((x0, x1, x2, x3)):
        b = xr[...]
        ssum_o[j] += jnp.sum(b, axis=0)
        ssq_o[j] += jnp.sum(b * b, axis=0)


_k4 = pl.pallas_call(
    _k4_body,
    grid=(NB,),
    in_specs=[pl.BlockSpec((RB, D), lambda i: (i, 0))] * 4,
    out_specs=(
        pl.BlockSpec((4, D), lambda i: (0, 0)),
        pl.BlockSpec((4, D), lambda i: (0, 0)),
    ),
    out_shape=(jax.ShapeDtypeStruct((4, D), jnp.float32),) * 2,
)


def _k5_body(x0, x1, x2, x3, ssum, ssq, gam, bet, wout, bout, out_o):
    mean = ssum[...] * (1.0 / N)
    var = ssq[...] * (1.0 / N) - mean * mean
    a = gam[...] * lax.rsqrt(var + BN_EPS)
    dvec = bet[...] - mean * a
    w = wout[...]
    base = bout[...]
    acc = jnp.zeros((RB, D), jnp.float32)
    for j, xr in enumerate((x0, x1, x2, x3)):
        base = base + jnp.dot(dvec[j][None, :], w[j],
                              preferred_element_type=jnp.float32)
        acc = acc + jnp.dot(xr[...] * a[j][None, :], w[j],
                            preferred_element_type=jnp.float32)
    out_o[...] = acc + base


_k5 = pl.pallas_call(
    _k5_body,
    grid=(NB,),
    in_specs=(
        [pl.BlockSpec((RB, D), lambda i: (i, 0))] * 4
        + [pl.BlockSpec((4, D), lambda i: (0, 0))] * 4
        + [pl.BlockSpec((4, D, D), lambda i: (0, 0, 0)),
           pl.BlockSpec((1, D), lambda i: (0, 0))]
    ),
    out_specs=pl.BlockSpec((RB, D), lambda i: (i, 0)),
    out_shape=jax.ShapeDtypeStruct((N, D), jnp.float32),
)


def _pad_idx(a, n, fill):
    return jnp.concatenate([a, jnp.full((n - E,), fill, jnp.int32)])


def kernel(x, edge_index, W0, b0, W1, b1, W2, b2, bn_gamma, bn_beta,
           W_out, b_out):
    ei = edge_index.astype(jnp.int32)
    src, dst = ei[0], ei[1]

    gA = _pad_idx(src, EPA, 0).reshape(NC, NS, NCH_A, C)
    sA = _pad_idx(dst, EPA, NPAD - 1).reshape(NC, NS, NCH_A, C)
    dstg = _pad_idx(dst, EPB, 0)
    gB = jnp.stack([dstg, dstg + N]).reshape(NC, NS, NCH_B, C)
    srcs = _pad_idx(src, EPB, NPAD - 1)
    sB = jnp.stack([srcs, srcs]).reshape(NC, NS, NCH_B, C)
    dsts = _pad_idx(dst, EPB, NPAD - 1)
    zrows = jnp.zeros((HD, D), jnp.float32)

    sc_gs_a, sc_gs_b, sc_counts = _sc_kernels()

    cs_parts, cd_parts = sc_counts(srcs.reshape(32, EC), dsts.reshape(32, EC))
    dinv_a, cntr_a, cnt_a = _k0(cs_parts, cd_parts)
    dinv = dinv_a[:N]
    cntr = cntr_a[:N]
    cnt = cnt_a[:N]

    Ws = (W0, W1, W2)
    bs = (b0, b1, b2)
    xcat = [x]
    for li in range(3):
        xs, xw = _k1(xcat[li], Ws[li], dinv)
        agg2 = sc_gs_a(xs, gA, sA, zrows)[:, :N]
        hh = _k2(agg2, xw, dinv, bs[li].reshape(1, D))
        s12 = sc_gs_b(hh.reshape(2 * N, D), gB, sB, zrows)[:, :N]
        xcat.append(_k3(s12, hh, xcat[li], cntr, cnt))

    ssum, ssq = _k4(*xcat)
    return _k5(*xcat, ssum, ssq, bn_gamma.reshape(4, D),
               bn_beta.reshape(4, D), W_out.reshape(4, D, D),
               b_out.reshape(1, D))


# speedup vs baseline: 1.1985x; 1.1985x over previous
import functools

import jax
import jax.numpy as jnp
from jax import lax
from jax.experimental import pallas as pl
from jax.experimental.pallas import tpu as pltpu
from jax.experimental.pallas import tpu_sc as plsc

N = 10000
E = 320000
D = 128
NPAD = 10240
NC = 2
NS = 16
C = 64
NSL = 4
NCH_A = 160
NCH_B = 316
EPA = NCH_A * C * NS * NC
EPB = NCH_B * C * NS
EC = EPB // 32
DRN = NPAD // NS
HD = DRN // 16
RB = 1000
NB = N // RB
BN_EPS = 1e-5


def _sc_gather_scatter_body(table, gidx, sidx, zrows, out, acc, *scr):
    gv = scr[0:NSL]
    sv = scr[NSL:2 * NSL]
    rv = scr[2 * NSL:3 * NSL]
    dv = scr[3 * NSL]
    sem_g = scr[3 * NSL + 1:4 * NSL + 1]
    sem_s = scr[4 * NSL + 1:5 * NSL + 1]
    sem_r = scr[5 * NSL + 1:6 * NSL + 1]
    sem_c = scr[6 * NSL + 1:7 * NSL + 1]
    nch = gidx.shape[2]
    c = lax.axis_index("c")
    s = lax.axis_index("s")
    pltpu.sync_copy(zrows, dv)
    for q in range(DRN // HD):
        pltpu.sync_copy(dv, acc.at[pl.ds(s * DRN + q * HD, HD)])
    plsc.subcore_barrier()

    for i in range(NSL):
        pltpu.async_copy(gidx.at[c, s, i], gv[i], sem_g[i])
        pltpu.async_copy(sidx.at[c, s, i], sv[i], sem_s[i])
    for i in range(NSL):
        pltpu.make_async_copy(gidx.at[c, s, i], gv[i], sem_g[i]).wait()
        pltpu.async_copy(table.at[gv[i]], rv[i], sem_r[i])

    def group(t, carry):
        j = NSL * t
        for i in range(NSL):
            pltpu.make_async_copy(table.at[gv[i]], rv[i], sem_r[i]).wait()
            pltpu.make_async_copy(sidx.at[c, s, j + i], sv[i], sem_s[i]).wait()
            pltpu.async_copy(rv[i], acc.at[sv[i]], sem_c[i], add=True)
        for i in range(NSL):
            pltpu.make_async_copy(rv[i], acc.at[sv[i]], sem_c[i]).wait()

            @pl.when(t < nch // NSL - 1)
            def _():
                pltpu.async_copy(gidx.at[c, s, j + NSL + i], gv[i], sem_g[i])
                pltpu.async_copy(sidx.at[c, s, j + NSL + i], sv[i], sem_s[i])
                pltpu.make_async_copy(
                    gidx.at[c, s, j + NSL + i], gv[i], sem_g[i]).wait()
                pltpu.async_copy(table.at[gv[i]], rv[i], sem_r[i])

        return carry

    lax.fori_loop(0, nch // NSL, group, 0)
    plsc.subcore_barrier()

    for q in range(DRN // HD):
        pltpu.sync_copy(acc.at[pl.ds(s * DRN + q * HD, HD)], dv)
        pltpu.sync_copy(dv, out.at[c, pl.ds(s * DRN + q * HD, HD)])


def _sc_counts_body(sidx, didx, out_s, out_d, sv, dv, cs, cd):
    c = lax.axis_index("c")
    s = lax.axis_index("s")
    wid = c * NS + s
    pltpu.sync_copy(sidx.at[wid], sv)
    pltpu.sync_copy(didx.at[wid], dv)

    zero = jnp.zeros((16,), jnp.float32)

    def zbody(i, carry):
        cs[pl.ds(i * 16, 16)] = zero
        cd[pl.ds(i * 16, 16)] = zero
        return carry

    lax.fori_loop(0, NPAD // 16, zbody, 0)

    ones = jnp.ones((16,), jnp.float32)

    def body(i, carry):
        iv = sv[pl.ds(i * 16, 16)]
        plsc.addupdate_scatter(cs, [iv], ones)
        jv = dv[pl.ds(i * 16, 16)]
        plsc.addupdate_scatter(cd, [jv], ones)
        return carry

    lax.fori_loop(0, EC // 16, body, 0)
    pltpu.sync_copy(cs, out_s.at[wid])
    pltpu.sync_copy(cd, out_d.at[wid])


@functools.lru_cache(maxsize=None)
def _sc_kernels():
    mesh = plsc.VectorSubcoreMesh(
        core_axis_name="c", subcore_axis_name="s",
        num_cores=NC, num_subcores=NS)
    params = pltpu.CompilerParams(needs_layout_passes=False)

    def make_gs(nch):
        return pl.kernel(
            _sc_gather_scatter_body,
            out_type=jax.ShapeDtypeStruct((NC, NPAD, D), jnp.float32),
            mesh=mesh,
            scratch_types=(
                [pltpu.VMEM_SHARED((NPAD, D), jnp.float32)]
                + [pltpu.VMEM((C,), jnp.int32)] * NSL
                + [pltpu.VMEM((C,), jnp.int32)] * NSL
                + [pltpu.VMEM((C, D), jnp.float32)] * NSL
                + [pltpu.VMEM((HD, D), jnp.float32)]
                + [pltpu.SemaphoreType.DMA] * (4 * NSL)
            ),
            compiler_params=params,
        )

    counts = pl.kernel(
        _sc_counts_body,
        out_type=(jax.ShapeDtypeStruct((NC * NS, NPAD), jnp.float32),
                  jax.ShapeDtypeStruct((NC * NS, NPAD), jnp.float32)),
        mesh=mesh,
        scratch_types=[
            pltpu.VMEM((EC,), jnp.int32),
            pltpu.VMEM((EC,), jnp.int32),
            pltpu.VMEM((NPAD,), jnp.float32),
            pltpu.VMEM((NPAD,), jnp.float32),
        ],
        compiler_params=params,
    )
    return make_gs(NCH_A), make_gs(NCH_B), counts


def _k0_body(csp, cdp, x, dinv_o, cntr_o, cnt_o, xd_o):
    cntr = jnp.sum(csp[...], axis=0)
    deg = jnp.sum(cdp[...], axis=0) + 1.0
    dinv = lax.rsqrt(deg)
    dinv_o[...] = dinv[:, None]
    cntr_o[...] = cntr[:, None]
    cnt_o[...] = jnp.maximum(cntr, 1.0)[:, None]
    xd_o[...] = x[...] * dinv[:N, None]


_k0 = pl.pallas_call(
    _k0_body,
    out_shape=(jax.ShapeDtypeStruct((NPAD, 1), jnp.float32),) * 3
    + (jax.ShapeDtypeStruct((N, D), jnp.float32),),
)


def _k2_body(agg2, x, w, dinv, b, hh_o):
    di = dinv[...]
    a = agg2[0] + agg2[1]
    aw = jnp.dot(a, w[...], preferred_element_type=jnp.float32)
    xw = jnp.dot(x[...], w[...], preferred_element_type=jnp.float32)
    h = jnp.maximum(di * aw + di * di * xw + b[...], 0.0)
    hh_o[0] = h
    hh_o[1] = h * h


_k2 = pl.pallas_call(
    _k2_body,
    grid=(NB,),
    in_specs=[
        pl.BlockSpec((NC, RB, D), lambda i: (0, i, 0)),
        pl.BlockSpec((RB, D), lambda i: (i, 0)),
        pl.BlockSpec((D, D), lambda i: (0, 0)),
        pl.BlockSpec((RB, 1), lambda i: (i, 0)),
        pl.BlockSpec((1, D), lambda i: (0, 0)),
    ],
    out_specs=pl.BlockSpec((2, RB, D), lambda i: (0, i, 0)),
    out_shape=jax.ShapeDtypeStruct((2, N, D), jnp.float32),
)


def _k3_body(s12, hh, x, cntr, cnt, dinv, xn_o, xd_o):
    h = hh[0]
    xb = x[...]
    s = cntr[...] * h * h - 2.0 * h * s12[0] + s12[1]
    gg = jnp.tanh(s / cnt[...])
    xn = (1.0 - gg) * xb + gg * h
    xn_o[...] = xn
    xd_o[...] = xn * dinv[...]


_k3 = pl.pallas_call(
    _k3_body,
    grid=(NB,),
    in_specs=[
        pl.BlockSpec((NC, RB, D), lambda i: (0, i, 0)),
        pl.BlockSpec((2, RB, D), lambda i: (0, i, 0)),
        pl.BlockSpec((RB, D), lambda i: (i, 0)),
        pl.BlockSpec((RB, 1), lambda i: (i, 0)),
        pl.BlockSpec((RB, 1), lambda i: (i, 0)),
        pl.BlockSpec((RB, 1), lambda i: (i, 0)),
    ],
    out_specs=(
        pl.BlockSpec((RB, D), lambda i: (i, 0)),
        pl.BlockSpec((RB, D), lambda i: (i, 0)),
    ),
    out_shape=(jax.ShapeDtypeStruct((N, D), jnp.float32),) * 2,
)


def _k4_body(x0, x1, x2, x3, ssum_o, ssq_o):
    i = pl.program_id(0)

    @pl.when(i == 0)
    def _():
        ssum_o[...] = jnp.zeros((4, D), jnp.float32)
        ssq_o[...] = jnp.zeros((4, D), jnp.float32)

    for j, xr in enumerate((x0, x1, x2, x3)):
        b = xr[...]
        ssum_o[j] += jnp.sum(b, axis=0)
        ssq_o[j] += jnp.sum(b * b, axis=0)


_k4 = pl.pallas_call(
    _k4_body,
    grid=(NB,),
    in_specs=[pl.BlockSpec((RB, D), lambda i: (i, 0))] * 4,
    out_specs=(
        pl.BlockSpec((4, D), lambda i: (0, 0)),
        pl.BlockSpec((4, D), lambda i: (0, 0)),
    ),
    out_shape=(jax.ShapeDtypeStruct((4, D), jnp.float32),) * 2,
)


def _k5_body(x0, x1, x2, x3, ssum, ssq, gam, bet, wout, bout, out_o):
    mean = ssum[...] * (1.0 / N)
    var = ssq[...] * (1.0 / N) - mean * mean
    a = gam[...] * lax.rsqrt(var + BN_EPS)
    dvec = bet[...] - mean * a
    w = wout[...]
    base = bout[...]
    acc = jnp.zeros((RB, D), jnp.float32)
    for j, xr in enumerate((x0, x1, x2, x3)):
        base = base + jnp.dot(dvec[j][None, :], w[j],
                              preferred_element_type=jnp.float32)
        acc = acc + jnp.dot(xr[...] * a[j][None, :], w[j],
                            preferred_element_type=jnp.float32)
    out_o[...] = acc + base


_k5 = pl.pallas_call(
    _k5_body,
    grid=(NB,),
    in_specs=(
        [pl.BlockSpec((RB, D), lambda i: (i, 0))] * 4
        + [pl.BlockSpec((4, D), lambda i: (0, 0))] * 4
        + [pl.BlockSpec((4, D, D), lambda i: (0, 0, 0)),
           pl.BlockSpec((1, D), lambda i: (0, 0))]
    ),
    out_specs=pl.BlockSpec((RB, D), lambda i: (i, 0)),
    out_shape=jax.ShapeDtypeStruct((N, D), jnp.float32),
)


def _pad_idx(a, n, fill):
    return jnp.concatenate([a, jnp.full((n - E,), fill, jnp.int32)])


def kernel(x, edge_index, W0, b0, W1, b1, W2, b2, bn_gamma, bn_beta,
           W_out, b_out):
    ei = edge_index.astype(jnp.int32)
    src, dst = ei[0], ei[1]

    gA = _pad_idx(src, EPA, 0).reshape(NC, NS, NCH_A, C)
    sA = _pad_idx(dst, EPA, NPAD - 1).reshape(NC, NS, NCH_A, C)
    dstg = _pad_idx(dst, EPB, 0)
    gB = jnp.stack([dstg, dstg + N]).reshape(NC, NS, NCH_B, C)
    srcs = _pad_idx(src, EPB, NPAD - 1)
    sB = jnp.stack([srcs, srcs]).reshape(NC, NS, NCH_B, C)
    dsts = _pad_idx(dst, EPB, NPAD - 1)
    zrows = jnp.zeros((HD, D), jnp.float32)

    sc_gs_a, sc_gs_b, sc_counts = _sc_kernels()

    cs_parts, cd_parts = sc_counts(srcs.reshape(32, EC), dsts.reshape(32, EC))
    dinv_a, cntr_a, cnt_a, xd = _k0(cs_parts, cd_parts, x)
    dinv = dinv_a[:N]
    cntr = cntr_a[:N]
    cnt = cnt_a[:N]

    Ws = (W0, W1, W2)
    bs = (b0, b1, b2)
    xcat = [x]
    for li in range(3):
        agg2 = sc_gs_a(xd, gA, sA, zrows)[:, :N]
        hh = _k2(agg2, xcat[li], Ws[li], dinv, bs[li].reshape(1, D))
        s12 = sc_gs_b(hh.reshape(2 * N, D), gB, sB, zrows)[:, :N]
        xn, xd = _k3(s12, hh, xcat[li], cntr, cnt, dinv)
        xcat.append(xn)

    ssum, ssq = _k4(*xcat)
    return _k5(*xcat, ssum, ssq, bn_gamma.reshape(4, D),
               bn_beta.reshape(4, D), W_out.reshape(4, D, D),
               b_out.reshape(1, D))


# speedup vs baseline: 1.2501x; 1.0430x over previous
import functools

import jax
import jax.numpy as jnp
from jax import lax
from jax.experimental import pallas as pl
from jax.experimental.pallas import tpu as pltpu
from jax.experimental.pallas import tpu_sc as plsc

N = 10000
E = 320000
D = 128
NPAD = 10240
NC = 2
NS = 16
C = 64
NSL = 4
NCH_A = 160
NCH_B = 316
EPA = NCH_A * C * NS * NC
EPB = NCH_B * C * NS
EC = EPB // 32
DRN = NPAD // NS
HD = DRN // 16
RB = 1000
NB = N // RB
BN_EPS = 1e-5


def _sc_gather_scatter_body(table, gidx, sidx, zrows, out, acc, *scr):
    gv = scr[0:NSL]
    sv = scr[NSL:2 * NSL]
    rv = scr[2 * NSL:3 * NSL]
    dv = scr[3 * NSL]
    sem_g = scr[3 * NSL + 1:4 * NSL + 1]
    sem_s = scr[4 * NSL + 1:5 * NSL + 1]
    sem_r = scr[5 * NSL + 1:6 * NSL + 1]
    sem_c = scr[6 * NSL + 1:7 * NSL + 1]
    nch = gidx.shape[2]
    c = lax.axis_index("c")
    s = lax.axis_index("s")
    pltpu.sync_copy(zrows, dv)
    for q in range(DRN // HD):
        pltpu.sync_copy(dv, acc.at[pl.ds(s * DRN + q * HD, HD)])
    plsc.subcore_barrier()

    for i in range(NSL):
        pltpu.async_copy(gidx.at[c, s, i], gv[i], sem_g[i])
        pltpu.async_copy(sidx.at[c, s, i], sv[i], sem_s[i])
    for i in range(NSL):
        pltpu.make_async_copy(gidx.at[c, s, i], gv[i], sem_g[i]).wait()
        pltpu.async_copy(table.at[gv[i]], rv[i], sem_r[i])

    def group(t, carry):
        j = NSL * t
        for i in range(NSL):
            pltpu.make_async_copy(table.at[gv[i]], rv[i], sem_r[i]).wait()
            pltpu.make_async_copy(sidx.at[c, s, j + i], sv[i], sem_s[i]).wait()
            pltpu.async_copy(rv[i], acc.at[sv[i]], sem_c[i], add=True)
        for i in range(NSL):
            pltpu.make_async_copy(rv[i], acc.at[sv[i]], sem_c[i]).wait()

            @pl.when(t < nch // NSL - 1)
            def _():
                pltpu.async_copy(gidx.at[c, s, j + NSL + i], gv[i], sem_g[i])
                pltpu.async_copy(sidx.at[c, s, j + NSL + i], sv[i], sem_s[i])
                pltpu.make_async_copy(
                    gidx.at[c, s, j + NSL + i], gv[i], sem_g[i]).wait()
                pltpu.async_copy(table.at[gv[i]], rv[i], sem_r[i])

        return carry

    lax.fori_loop(0, nch // NSL, group, 0)
    plsc.subcore_barrier()

    for q in range(DRN // HD):
        pltpu.sync_copy(acc.at[pl.ds(s * DRN + q * HD, HD)], dv)
        pltpu.sync_copy(dv, out.at[c, pl.ds(s * DRN + q * HD, HD)])


def _sc_counts_body(sidx, didx, out_s, out_d, sv, dv, cs, cd):
    c = lax.axis_index("c")
    s = lax.axis_index("s")
    wid = c * NS + s
    pltpu.sync_copy(sidx.at[wid], sv)
    pltpu.sync_copy(didx.at[wid], dv)

    zero = jnp.zeros((16,), jnp.float32)

    def zbody(i, carry):
        cs[pl.ds(i * 16, 16)] = zero
        cd[pl.ds(i * 16, 16)] = zero
        return carry

    lax.fori_loop(0, NPAD // 16, zbody, 0)

    ones = jnp.ones((16,), jnp.float32)

    def body(i, carry):
        iv = sv[pl.ds(i * 16, 16)]
        plsc.addupdate_scatter(cs, [iv], ones)
        jv = dv[pl.ds(i * 16, 16)]
        plsc.addupdate_scatter(cd, [jv], ones)
        return carry

    lax.fori_loop(0, EC // 16, body, 0)
    pltpu.sync_copy(cs, out_s.at[wid])
    pltpu.sync_copy(cd, out_d.at[wid])


@functools.lru_cache(maxsize=None)
def _sc_kernels():
    mesh = plsc.VectorSubcoreMesh(
        core_axis_name="c", subcore_axis_name="s",
        num_cores=NC, num_subcores=NS)
    params = pltpu.CompilerParams(needs_layout_passes=False)

    def make_gs(nch):
        return pl.kernel(
            _sc_gather_scatter_body,
            out_type=jax.ShapeDtypeStruct((NC, NPAD, D), jnp.float32),
            mesh=mesh,
            scratch_types=(
                [pltpu.VMEM_SHARED((NPAD, D), jnp.float32)]
                + [pltpu.VMEM((C,), jnp.int32)] * NSL
                + [pltpu.VMEM((C,), jnp.int32)] * NSL
                + [pltpu.VMEM((C, D), jnp.float32)] * NSL
                + [pltpu.VMEM((HD, D), jnp.float32)]
                + [pltpu.SemaphoreType.DMA] * (4 * NSL)
            ),
            compiler_params=params,
        )

    counts = pl.kernel(
        _sc_counts_body,
        out_type=(jax.ShapeDtypeStruct((NC * NS, NPAD), jnp.float32),
                  jax.ShapeDtypeStruct((NC * NS, NPAD), jnp.float32)),
        mesh=mesh,
        scratch_types=[
            pltpu.VMEM((EC,), jnp.int32),
            pltpu.VMEM((EC,), jnp.int32),
            pltpu.VMEM((NPAD,), jnp.float32),
            pltpu.VMEM((NPAD,), jnp.float32),
        ],
        compiler_params=params,
    )
    return make_gs(NCH_A), make_gs(NCH_B), counts


def _k0_body(csp, cdp, dinv_o, cntr_o, cnt_o):
    cntr = jnp.sum(csp[...], axis=0)
    deg = jnp.sum(cdp[...], axis=0) + 1.0
    dinv_o[...] = lax.rsqrt(deg)[:, None]
    cntr_o[...] = cntr[:, None]
    cnt_o[...] = jnp.maximum(cntr, 1.0)[:, None]


_k0 = pl.pallas_call(
    _k0_body,
    out_shape=(jax.ShapeDtypeStruct((NPAD, 1), jnp.float32),) * 3,
)


def _k1_body(x, w, dinv, xs_o, xw_o):
    xw = jnp.dot(x[...], w[...], preferred_element_type=jnp.float32)
    xw_o[...] = xw
    xs_o[...] = xw * dinv[...]


_k1 = pl.pallas_call(
    _k1_body,
    grid=(NB,),
    in_specs=[
        pl.BlockSpec((RB, D), lambda i: (i, 0)),
        pl.BlockSpec((D, D), lambda i: (0, 0)),
        pl.BlockSpec((RB, 1), lambda i: (i, 0)),
    ],
    out_specs=(
        pl.BlockSpec((RB, D), lambda i: (i, 0)),
        pl.BlockSpec((RB, D), lambda i: (i, 0)),
    ),
    out_shape=(jax.ShapeDtypeStruct((N, D), jnp.float32),) * 2,
)


def _k2_body(agg2, xw, dinv, b, hh_o):
    di = dinv[...]
    a = agg2[0] + agg2[1]
    h = jnp.maximum(di * a + di * di * xw[...] + b[...], 0.0)
    hh_o[0] = h
    hh_o[1] = h * h


_k2 = pl.pallas_call(
    _k2_body,
    grid=(NB,),
    in_specs=[
        pl.BlockSpec((NC, RB, D), lambda i: (0, i, 0)),
        pl.BlockSpec((RB, D), lambda i: (i, 0)),
        pl.BlockSpec((RB, 1), lambda i: (i, 0)),
        pl.BlockSpec((1, D), lambda i: (0, 0)),
    ],
    out_specs=pl.BlockSpec((2, RB, D), lambda i: (0, i, 0)),
    out_shape=jax.ShapeDtypeStruct((2, N, D), jnp.float32),
)


def _k3_body(s12, hh, x, cntr, cnt, xn_o):
    h = hh[0]
    xb = x[...]
    s = cntr[...] * h * h - 2.0 * h * s12[0] + s12[1]
    gg = jnp.tanh(s / cnt[...])
    xn_o[...] = (1.0 - gg) * xb + gg * h


_k3 = pl.pallas_call(
    _k3_body,
    grid=(NB,),
    in_specs=[
        pl.BlockSpec((NC, RB, D), lambda i: (0, i, 0)),
        pl.BlockSpec((2, RB, D), lambda i: (0, i, 0)),
        pl.BlockSpec((RB, D), lambda i: (i, 0)),
        pl.BlockSpec((RB, 1), lambda i: (i, 0)),
        pl.BlockSpec((RB, 1), lambda i: (i, 0)),
    ],
    out_specs=pl.BlockSpec((RB, D), lambda i: (i, 0)),
    out_shape=jax.ShapeDtypeStruct((N, D), jnp.float32),
)


def _k4_body(x0, x1, x2, x3, ssum_o, ssq_o):
    i = pl.program_id(0)

    @pl.when(i == 0)
    def _():
        ssum_o[...] = jnp.zeros((4, D), jnp.float32)
        ssq_o[...] = jnp.zeros((4, D), jnp.float32)

    for j, xr in enumerate((x0, x1, x2, x3)):
        b = xr[...]
        ssum_o[j] += jnp.sum(b, axis=0)
        ssq_o[j] += jnp.sum(b * b, axis=0)


_k4 = pl.pallas_call(
    _k4_body,
    grid=(NB,),
    in_specs=[pl.BlockSpec((RB, D), lambda i: (i, 0))] * 4,
    out_specs=(
        pl.BlockSpec((4, D), lambda i: (0, 0)),
        pl.BlockSpec((4, D), lambda i: (0, 0)),
    ),
    out_shape=(jax.ShapeDtypeStruct((4, D), jnp.float32),) * 2,
)


def _k5_body(x0, x1, x2, x3, ssum, ssq, gam, bet, wout, bout, out_o):
    mean = ssum[...] * (1.0 / N)
    var = ssq[...] * (1.0 / N) - mean * mean
    a = gam[...] * lax.rsqrt(var + BN_EPS)
    dvec = bet[...] - mean * a
    w = wout[...]
    base = bout[...]
    acc = jnp.zeros((RB, D), jnp.float32)
    for j, xr in enumerate((x0, x1, x2, x3)):
        base = base + jnp.dot(dvec[j][None, :], w[j],
                              preferred_element_type=jnp.float32)
        acc = acc + jnp.dot(xr[...] * a[j][None, :], w[j],
                            preferred_element_type=jnp.float32)
    out_o[...] = acc + base


_k5 = pl.pallas_call(
    _k5_body,
    grid=(NB,),
    in_specs=(
        [pl.BlockSpec((RB, D), lambda i: (i, 0))] * 4
        + [pl.BlockSpec((4, D), lambda i: (0, 0))] * 4
        + [pl.BlockSpec((4, D, D), lambda i: (0, 0, 0)),
           pl.BlockSpec((1, D), lambda i: (0, 0))]
    ),
    out_specs=pl.BlockSpec((RB, D), lambda i: (i, 0)),
    out_shape=jax.ShapeDtypeStruct((N, D), jnp.float32),
)


def _pad_idx(a, n, fill):
    return jnp.concatenate([a, jnp.full((n - E,), fill, jnp.int32)])


def kernel(x, edge_index, W0, b0, W1, b1, W2, b2, bn_gamma, bn_beta,
           W_out, b_out):
    ei = edge_index.astype(jnp.int32)
    src, dst = ei[0], ei[1]

    gA = _pad_idx(src, EPA, 0).reshape(NC, NS, NCH_A, C)
    sA = _pad_idx(dst, EPA, NPAD - 1).reshape(NC, NS, NCH_A, C)
    dstg = _pad_idx(dst, EPB, 0)
    gB = jnp.stack([dstg, dstg + N]).reshape(NC, NS, NCH_B, C)
    srcs = _pad_idx(src, EPB, NPAD - 1)
    sB = jnp.stack([srcs, srcs]).reshape(NC, NS, NCH_B, C)
    dsts = _pad_idx(dst, EPB, NPAD - 1)
    zrows = jnp.zeros((HD, D), jnp.float32)

    sc_gs_a, sc_gs_b, sc_counts = _sc_kernels()

    cs_parts, cd_parts = sc_counts(srcs.reshape(32, EC), dsts.reshape(32, EC))
    dinv_a, cntr_a, cnt_a = _k0(cs_parts, cd_parts)
    dinv = dinv_a[:N]
    cntr = cntr_a[:N]
    cnt = cnt_a[:N]

    Ws = (W0, W1, W2)
    bs = (b0, b1, b2)
    xcat = [x]
    for li in range(3):
        xs, xw = _k1(xcat[li], Ws[li], dinv)
        agg2 = sc_gs_a(xs, gA, sA, zrows)[:, :N]
        hh = _k2(agg2, xw, dinv, bs[li].reshape(1, D))
        s12 = sc_gs_b(hh.reshape(2 * N, D), gB, sB, zrows)[:, :N]
        xcat.append(_k3(s12, hh, xcat[li], cntr, cnt))

    ssum, ssq = _k4(*xcat)
    return _k5(*xcat, ssum, ssq, bn_gamma.reshape(4, D),
               bn_beta.reshape(4, D), W_out.reshape(4, D, D),
               b_out.reshape(1, D))
